# initial kernel scaffold (unmeasured)
import jax
import jax.numpy as jnp
from jax import lax
from jax.experimental import pallas as pl
from jax.experimental.pallas import tpu as pltpu


def kernel(
    x,
):
    def body(*refs):
        pass

    out_shape = jax.ShapeDtypeStruct(..., jnp.float32)
    return pl.pallas_call(body, out_shape=out_shape)(...)


# baseline (device time: 10359156 ns/iter reference)
import jax
import jax.numpy as jnp
from jax import lax
from jax.experimental import pallas as pl
from jax.experimental.pallas import tpu as pltpu

N_DEV = 16
CHUNK = 128


def _cmpx_dyn(v, j, asc, roll):
    rows = v.shape[0]
    i = lax.broadcasted_iota(jnp.int32, (rows, 1), 0)
    first = (i & j) == 0
    up = roll(v, rows - j, 0)
    dn = roll(v, j, 0)
    partner = jnp.where(first, up, dn)
    lo = jnp.minimum(v, partner)
    hi = jnp.maximum(v, partner)
    keep_lo = jnp.logical_not(jnp.logical_xor(first, asc))
    return jnp.where(keep_lo, lo, hi)


def _bitonic_merge(v, asc, roll, log_rows):
    def step(t, v):
        j = jnp.int32(1) << (log_rows - 1 - t)
        return _cmpx_dyn(v, j, asc, roll)
    return lax.fori_loop(0, log_rows, step, v)


def _bitonic_sort(v, desc_dev, roll, log_rows):
    rows = v.shape[0]
    i = lax.broadcasted_iota(jnp.int32, (rows, 1), 0)

    def outer(lgk, v):
        k = jnp.int32(1) << lgk
        asc = jnp.logical_xor((i & k) == 0, desc_dev)

        def inner(t, v):
            j = jnp.int32(1) << (lgk - 1 - t)
            return _cmpx_dyn(v, j, asc, roll)

        return lax.fori_loop(0, lgk, inner, v)

    return lax.fori_loop(1, log_rows + 1, outer, v)


def kernel(x):
    m, n = x.shape
    log_m = m.bit_length() - 1
    n_chunks = n // CHUNK
    x = x.astype(jnp.bfloat16)

    def body(x_ref, out_ref, comm_ref, send_sem, recv_sem, credit_sem):
        my = lax.axis_index("i")
        desc_dev = (my % 2) == 1
        is_edge = jnp.logical_or(my == 0, my == N_DEV - 1)

        def sort_chunk(c, carry):
            cs = pl.ds(c * CHUNK, CHUNK)
            out_ref[:, cs] = _bitonic_sort(x_ref[:, cs], desc_dev,
                                           pltpu.roll, log_m)
            return carry

        lax.fori_loop(0, n_chunks, sort_chunk, jnp.int32(0))

        def phase(p, carry):
            even = (p % 2) == 0
            partner = jnp.where(
                even, my ^ 1,
                jnp.where(my % 2 == 0, my - 1, my + 1),
            )
            active = jnp.logical_or(even, jnp.logical_not(is_edge))
            am_lower = my < partner

            @pl.when(active)
            def _():
                @pl.when(p >= 1)
                def _():
                    pl.semaphore_wait(credit_sem, 1)

                rdma = pltpu.make_async_remote_copy(
                    src_ref=out_ref,
                    dst_ref=comm_ref,
                    send_sem=send_sem,
                    recv_sem=recv_sem,
                    device_id=(partner,),
                    device_id_type=pl.DeviceIdType.MESH,
                )
                rdma.start()
                rdma.wait()

                def merge_chunk(c, carry):
                    cs = pl.ds(c * CHUNK, CHUNK)
                    rv = comm_ref[:, cs]
                    cur = out_ref[:, cs]
                    merged = jnp.where(am_lower, jnp.minimum(cur, rv),
                                       jnp.maximum(cur, rv))
                    merged = _bitonic_merge(merged,
                                            jnp.logical_not(desc_dev),
                                            pltpu.roll, log_m)
                    out_ref[:, cs] = merged
                    return carry

                lax.fori_loop(0, n_chunks, merge_chunk, jnp.int32(0))

                nxt = jnp.where(is_edge, partner, 2 * my - partner)
                grant = jnp.where(is_edge, p <= N_DEV - 3, p <= N_DEV - 2)

                @pl.when(grant)
                def _():
                    pl.semaphore_signal(
                        credit_sem, inc=1,
                        device_id=(nxt,),
                        device_id_type=pl.DeviceIdType.MESH,
                    )

            return carry

        lax.fori_loop(0, N_DEV, phase, jnp.int32(0))

        def final_chunk(c, carry):
            cs = pl.ds(c * CHUNK, CHUNK)
            out_ref[:, cs] = _bitonic_merge(out_ref[:, cs], jnp.bool_(True),
                                            pltpu.roll, log_m)
            return carry

        lax.fori_loop(0, n_chunks, final_chunk, jnp.int32(0))

    return pl.pallas_call(
        body,
        out_shape=jax.ShapeDtypeStruct((m, n), jnp.bfloat16),
        in_specs=[pl.BlockSpec(memory_space=pltpu.VMEM)],
        out_specs=pl.BlockSpec(memory_space=pltpu.VMEM),
        scratch_shapes=[
            pltpu.VMEM((m, n), jnp.bfloat16),
            pltpu.SemaphoreType.DMA,
            pltpu.SemaphoreType.DMA,
            pltpu.SemaphoreType.REGULAR,
        ],
    )(x)


# device time: 4856002 ns/iter; 2.1333x vs baseline; 2.1333x over previous
import jax
import jax.numpy as jnp
from jax import lax
from jax.experimental import pallas as pl
from jax.experimental.pallas import tpu as pltpu

N_DEV = 16
CHUNK = 128


def _cmpx_reshape(v, j):
    rows, w = v.shape
    v4 = v.reshape(rows // (2 * j), 2, j, w)
    a = v4[:, 0]
    b = v4[:, 1]
    lo = jnp.minimum(a, b)
    hi = jnp.maximum(a, b)
    return jnp.stack([lo, hi], axis=1).reshape(rows, w)


def _cmpx_roll(v, j, asc):
    rows = v.shape[0]
    i = lax.broadcasted_iota(jnp.int32, (rows, 1), 0)
    first = (i & j) == 0
    up = pltpu.roll(v, rows - j, 0)
    dn = pltpu.roll(v, j, 0)
    partner = jnp.where(first, up, dn)
    lo = jnp.minimum(v, partner)
    hi = jnp.maximum(v, partner)
    keep_lo = jnp.logical_not(jnp.logical_xor(first, asc))
    return jnp.where(keep_lo, lo, hi)


def _merge_asc(v):
    rows = v.shape[0]
    j = rows // 2
    while j >= 8:
        v = _cmpx_reshape(v, j)
        j //= 2
    while j >= 1:
        v = _cmpx_roll(v, j, True)
        j //= 2
    return v


def _sort_asc(v, log_rows):
    rows = v.shape[0]
    i = lax.broadcasted_iota(jnp.int32, (rows, 1), 0)

    def outer(lgk, v):
        k = jnp.int32(1) << lgk
        asc = (i & k) == 0

        def inner(t, v):
            j = jnp.int32(1) << (lgk - 1 - t)
            return _cmpx_roll(v, j, asc)

        return lax.fori_loop(0, lgk, inner, v)

    return lax.fori_loop(1, log_rows + 1, outer, v)


def kernel(x):
    m, n = x.shape
    log_m = m.bit_length() - 1
    n_chunks = n // CHUNK
    x = x.astype(jnp.bfloat16)

    def body(x_ref, out_ref, comm_ref, send_sem, recv_sem, credit_sem):
        my = lax.axis_index("i")
        desc_dev = (my % 2) == 1
        is_edge = jnp.logical_or(my == 0, my == N_DEV - 1)

        def sort_chunk(c, carry):
            cs = pl.ds(c * CHUNK, CHUNK)
            v = x_ref[:, cs]
            v = jnp.where(desc_dev, -v, v)
            out_ref[:, cs] = _sort_asc(v, log_m)
            return carry

        lax.fori_loop(0, n_chunks, sort_chunk, jnp.int32(0))

        def phase(p, carry):
            even = (p % 2) == 0
            partner = jnp.where(
                even, my ^ 1,
                jnp.where(my % 2 == 0, my - 1, my + 1),
            )
            active = jnp.logical_or(even, jnp.logical_not(is_edge))
            am_lower = my < partner
            take_min = jnp.logical_xor(am_lower, desc_dev)

            @pl.when(active)
            def _():
                @pl.when(p >= 1)
                def _():
                    pl.semaphore_wait(credit_sem, 1)

                rdma = pltpu.make_async_remote_copy(
                    src_ref=out_ref,
                    dst_ref=comm_ref,
                    send_sem=send_sem,
                    recv_sem=recv_sem,
                    device_id=(partner,),
                    device_id_type=pl.DeviceIdType.MESH,
                )
                rdma.start()
                rdma.wait()

                def merge_chunk(c, carry):
                    cs = pl.ds(c * CHUNK, CHUNK)
                    rvn = -comm_ref[:, cs]
                    cur = out_ref[:, cs]
                    merged = jnp.where(take_min, jnp.minimum(cur, rvn),
                                       jnp.maximum(cur, rvn))
                    out_ref[:, cs] = _merge_asc(merged)
                    return carry

                lax.fori_loop(0, n_chunks, merge_chunk, jnp.int32(0))

                nxt = jnp.where(is_edge, partner, 2 * my - partner)
                grant = jnp.where(is_edge, p <= N_DEV - 3, p <= N_DEV - 2)

                @pl.when(grant)
                def _():
                    pl.semaphore_signal(
                        credit_sem, inc=1,
                        device_id=(nxt,),
                        device_id_type=pl.DeviceIdType.MESH,
                    )

            return carry

        lax.fori_loop(0, N_DEV, phase, jnp.int32(0))

        def final_chunk(c, carry):
            cs = pl.ds(c * CHUNK, CHUNK)
            z = out_ref[:, cs]
            z = jnp.where(desc_dev, -z, z)
            out_ref[:, cs] = _merge_asc(z)
            return carry

        lax.fori_loop(0, n_chunks, final_chunk, jnp.int32(0))

    return pl.pallas_call(
        body,
        out_shape=jax.ShapeDtypeStruct((m, n), jnp.bfloat16),
        in_specs=[pl.BlockSpec(memory_space=pltpu.VMEM)],
        out_specs=pl.BlockSpec(memory_space=pltpu.VMEM),
        scratch_shapes=[
            pltpu.VMEM((m, n), jnp.bfloat16),
            pltpu.SemaphoreType.DMA,
            pltpu.SemaphoreType.DMA,
            pltpu.SemaphoreType.REGULAR,
        ],
    )(x)


# device time: 2062978 ns/iter; 5.0215x vs baseline; 2.3539x over previous
import jax
import jax.numpy as jnp
from jax import lax
from jax.experimental import pallas as pl
from jax.experimental.pallas import tpu as pltpu

N_DEV = 16
CHUNK = 128


def _cmpx_reshape(v, j):
    rows, w = v.shape
    v4 = v.reshape(rows // (2 * j), 2, j, w)
    a = v4[:, 0]
    b = v4[:, 1]
    lo = jnp.minimum(a, b)
    hi = jnp.maximum(a, b)
    return jnp.stack([lo, hi], axis=1).reshape(rows, w)


def _cmpx_roll(v, j, asc):
    rows = v.shape[0]
    i = lax.broadcasted_iota(jnp.int32, (rows, 1), 0)
    first = (i & j) == 0
    up = pltpu.roll(v, rows - j, 0)
    dn = pltpu.roll(v, j, 0)
    partner = jnp.where(first, up, dn)
    lo = jnp.minimum(v, partner)
    hi = jnp.maximum(v, partner)
    keep_lo = jnp.logical_not(jnp.logical_xor(first, asc))
    return jnp.where(keep_lo, lo, hi)


def _merge_asc(v):
    rows = v.shape[0]
    j = rows // 2
    while j >= 8:
        v = _cmpx_reshape(v, j)
        j //= 2
    while j >= 1:
        v = _cmpx_roll(v, j, True)
        j //= 2
    return v


def _cmpx_reshape_dir(v, j, k):
    rows, w = v.shape
    g_sz = rows // (2 * j)
    v4 = v.reshape(g_sz, 2, j, w)
    a = v4[:, 0]
    b = v4[:, 1]
    lo = jnp.minimum(a, b)
    hi = jnp.maximum(a, b)
    g = lax.broadcasted_iota(jnp.int32, (g_sz, 1, 1), 0)
    asc = (g & (k // (2 * j))) == 0
    t0 = jnp.where(asc, lo, hi)
    t1 = jnp.where(asc, hi, lo)
    return jnp.stack([t0, t1], axis=1).reshape(rows, w)


def _sort_asc(v, log_rows):
    rows = v.shape[0]
    i = lax.broadcasted_iota(jnp.int32, (rows, 1), 0)
    k = 2
    while k <= rows:
        j = k // 2
        while j >= 1:
            if j >= 8:
                v = _cmpx_reshape_dir(v, j, k)
            else:
                v = _cmpx_roll(v, j, (i & k) == 0)
            j //= 2
        k *= 2
    return v


def kernel(x):
    m, n = x.shape
    log_m = m.bit_length() - 1
    n_chunks = n // CHUNK
    x = x.astype(jnp.bfloat16)

    def body(x_ref, out_ref, comm_ref, send_sems, recv_sems, intra_sem,
             inter_sem):
        my = lax.axis_index("i")
        desc_dev = (my % 2) == 1
        is_edge = jnp.logical_or(my == 0, my == N_DEV - 1)

        def sort_chunk(c, carry):
            cs = pl.ds(c * CHUNK, CHUNK)
            v = x_ref[:, cs]
            v = jnp.where(desc_dev, -v, v)
            out_ref[:, cs] = _sort_asc(v, log_m)
            return carry

        lax.fori_loop(0, n_chunks, sort_chunk, jnp.int32(0))

        def phase(p, carry):
            even = (p % 2) == 0
            partner = jnp.where(
                even, my ^ 1,
                jnp.where(my % 2 == 0, my - 1, my + 1),
            )
            active = jnp.logical_or(even, jnp.logical_not(is_edge))
            am_lower = my < partner
            take_min = jnp.logical_xor(am_lower, desc_dev)

            nxt = jnp.where(is_edge, partner, 2 * my - partner)
            nxt_exists = jnp.where(is_edge, p <= N_DEV - 3, p <= N_DEV - 2)

            @pl.when(active)
            def _():
                def mk(c):
                    slot = c % 2
                    return pltpu.make_async_remote_copy(
                        src_ref=out_ref.at[:, pl.ds(c * CHUNK, CHUNK)],
                        dst_ref=comm_ref.at[slot],
                        send_sem=send_sems.at[slot],
                        recv_sem=recv_sems.at[slot],
                        device_id=(partner,),
                        device_id_type=pl.DeviceIdType.MESH,
                    )

                def consume(cc):
                    mk(cc).wait_send()
                    mk(cc).wait_recv()
                    cs = pl.ds(cc * CHUNK, CHUNK)
                    rvn = -comm_ref[cc % 2]
                    cur = out_ref[:, cs]
                    merged = jnp.where(take_min, jnp.minimum(cur, rvn),
                                       jnp.maximum(cur, rvn))
                    out_ref[:, cs] = _merge_asc(merged)
                    intra = cc <= n_chunks - 3

                    @pl.when(intra)
                    def _():
                        pl.semaphore_signal(
                            intra_sem, inc=1,
                            device_id=(partner,),
                            device_id_type=pl.DeviceIdType.MESH,
                        )

                    @pl.when(jnp.logical_and(jnp.logical_not(intra),
                                             nxt_exists))
                    def _():
                        pl.semaphore_signal(
                            inter_sem, inc=1,
                            device_id=(nxt,),
                            device_id_type=pl.DeviceIdType.MESH,
                        )

                def chunk_iter(c, carry2):
                    @pl.when(c >= 2)
                    def _():
                        pl.semaphore_wait(intra_sem, 1)

                    @pl.when(jnp.logical_and(c < 2, p >= 1))
                    def _():
                        pl.semaphore_wait(inter_sem, 1)

                    mk(c).start()

                    @pl.when(c >= 1)
                    def _():
                        consume(c - 1)

                    return carry2

                lax.fori_loop(0, n_chunks, chunk_iter, jnp.int32(0))
                consume(jnp.int32(n_chunks - 1))

            return carry

        lax.fori_loop(0, N_DEV, phase, jnp.int32(0))

        def final_chunk(c, carry):
            cs = pl.ds(c * CHUNK, CHUNK)
            z = out_ref[:, cs]
            z = jnp.where(desc_dev, -z, z)
            out_ref[:, cs] = _merge_asc(z)
            return carry

        lax.fori_loop(0, n_chunks, final_chunk, jnp.int32(0))

    return pl.pallas_call(
        body,
        out_shape=jax.ShapeDtypeStruct((m, n), jnp.bfloat16),
        in_specs=[pl.BlockSpec(memory_space=pltpu.VMEM)],
        out_specs=pl.BlockSpec(memory_space=pltpu.VMEM),
        scratch_shapes=[
            pltpu.VMEM((2, m, CHUNK), jnp.bfloat16),
            pltpu.SemaphoreType.DMA((2,)),
            pltpu.SemaphoreType.DMA((2,)),
            pltpu.SemaphoreType.REGULAR,
            pltpu.SemaphoreType.REGULAR,
        ],
    )(x)
